# Initial kernel scaffold; baseline (speedup 1.0000x reference)
import jax
import jax.numpy as jnp
from jax.experimental import pallas as pl


def kernel(node_feats, edge_index, etype, norm, W, bias, Wres, bres, gamma, beta):
    raise NotImplementedError("write your pallas kernel here")


# speedup vs baseline: 11.7358x; 11.7358x over previous
import functools

import jax
import jax.numpy as jnp
from jax import lax
from jax.experimental import pallas as pl
from jax.experimental.pallas import tpu as pltpu
from jax.experimental.pallas import tpu_sc as plsc

_EPS = 1e-5
_NC = 2
_NS = 16
_LANES = 16


def _xw_body(nrels, x_ref, w_ref, bres_ref, out_ref):
    r = pl.program_id(0)
    t = jnp.dot(x_ref[...], w_ref[0], preferred_element_type=jnp.float32)
    t = jnp.where(r == nrels, jnp.maximum(t + bres_ref[...], 0.0), t)
    out_ref[0] = t


def _compute_xw(x, w_all, bres, nrels, block_n):
    n, d_in = x.shape
    rp1, _, d_out = w_all.shape
    grid = (rp1, n // block_n)
    return pl.pallas_call(
        functools.partial(_xw_body, nrels),
        grid=grid,
        in_specs=[
            pl.BlockSpec((block_n, d_in), lambda r, b: (b, 0)),
            pl.BlockSpec((1, d_in, d_out), lambda r, b: (r, 0, 0)),
            pl.BlockSpec((1, d_out), lambda r, b: (0, 0)),
        ],
        out_specs=pl.BlockSpec((1, block_n, d_out), lambda r, b: (r, b, 0)),
        out_shape=jax.ShapeDtypeStruct((rp1, n, d_out), jnp.float32),
    )(x, w_all, bres.reshape(1, d_out))


def _gidx_body(n, et_ref, src_ref, out_ref):
    out_ref[...] = et_ref[...] * n + src_ref[...]


def _compute_gidx(etype, src, n):
    e = etype.shape[0]
    cols = 128
    rows = e // cols
    out = pl.pallas_call(
        functools.partial(_gidx_body, n),
        out_shape=jax.ShapeDtypeStruct((rows, cols), jnp.int32),
    )(etype.reshape(rows, cols), src.reshape(rows, cols))
    return out.reshape(e)


def _make_sc_edge_kernel(n, d, e, chunk):
    nw = _NC * _NS
    epw = e // nw
    steps = epw // chunk
    unit = 80
    units = n // unit
    rounds = (units + _NS - 1) // _NS

    def body(gidx_hbm, dst_hbm, norm_hbm, xw_hbm, out_hbm,
             idx_v, dst_v, norm_v, rows_v, zbuf_v, agg_sh, sem):
        c = lax.axis_index("c")
        s = lax.axis_index("s")

        zv = jnp.zeros((_LANES,), jnp.float32)

        def zrow(i, _):
            for j in range(d // _LANES):
                zbuf_v[i, pl.ds(j * _LANES, _LANES)] = zv
            return 0

        lax.fori_loop(0, unit, zrow, 0)
        for k in range(rounds):
            u = s + k * _NS

            @pl.when(u < units)
            def _():
                pltpu.sync_copy(zbuf_v, agg_sh.at[pl.ds(u * unit, unit)])
        plsc.subcore_barrier()

        wid = s * _NC + c
        ebase0 = wid * epw

        def step(t, _):
            ebase = ebase0 + t * chunk
            pltpu.sync_copy(gidx_hbm.at[pl.ds(ebase, chunk)], idx_v)
            pltpu.sync_copy(dst_hbm.at[pl.ds(ebase, chunk)], dst_v)
            pltpu.sync_copy(norm_hbm.at[pl.ds(ebase, chunk)], norm_v)
            pltpu.async_copy(xw_hbm.at[idx_v], rows_v, sem).wait()
            for g in range(chunk // _LANES):
                nv = norm_v[pl.ds(g * _LANES, _LANES)]
                for i in range(_LANES):
                    ee = g * _LANES + i
                    nb = nv[i]
                    for j in range(d // _LANES):
                        sl = pl.ds(j * _LANES, _LANES)
                        rows_v[ee, sl] = rows_v[ee, sl] * nb
            pltpu.sync_copy(rows_v, agg_sh.at[dst_v], add=True)
            return 0

        lax.fori_loop(0, steps, step, 0)
        plsc.subcore_barrier()

        for k in range(rounds):
            u = s + k * _NS

            @pl.when(u < units)
            def _():
                pltpu.sync_copy(agg_sh.at[pl.ds(u * unit, unit)], zbuf_v)
                pltpu.sync_copy(zbuf_v, out_hbm.at[pl.ds(c * n + u * unit,
                                                         unit)])

    mesh = plsc.VectorSubcoreMesh(core_axis_name="c", subcore_axis_name="s")
    return pl.kernel(
        body,
        out_type=jax.ShapeDtypeStruct((_NC * n, d), jnp.float32),
        mesh=mesh,
        scratch_types=[
            pltpu.VMEM((chunk,), jnp.int32),
            pltpu.VMEM((chunk,), jnp.int32),
            pltpu.VMEM((chunk,), jnp.float32),
            pltpu.VMEM((chunk, d), jnp.float32),
            pltpu.VMEM((unit, d), jnp.float32),
            pltpu.VMEM_SHARED((n, d), jnp.float32),
            pltpu.SemaphoreType.DMA,
        ],
    )


def _bn_body(n, eps, part_ref, res_ref, bias_ref, gamma_ref, beta_ref,
             out_ref):
    agg = part_ref[:n] + part_ref[n:]
    h = jnp.maximum(agg + bias_ref[...], 0.0)
    new = h + res_ref[...]
    mean = jnp.mean(new, axis=0, keepdims=True)
    var = jnp.mean((new - mean) * (new - mean), axis=0, keepdims=True)
    inv = lax.rsqrt(var + eps)
    out_ref[...] = (new - mean) * (inv * gamma_ref[...]) + beta_ref[...]


def _combine_bn(part, res, bias, gamma, beta, n, d):
    return pl.pallas_call(
        functools.partial(_bn_body, n, _EPS),
        out_shape=jax.ShapeDtypeStruct((n, d), jnp.float32),
    )(part, res, bias.reshape(1, d), gamma.reshape(1, d), beta.reshape(1, d))


def kernel(node_feats, edge_index, etype, norm, W, bias, Wres, bres, gamma,
           beta):
    n, d_in = node_feats.shape
    nrels, _, d_out = W.shape
    e = etype.shape[0]

    src = edge_index[0]
    dst = edge_index[1]

    w_all = jnp.concatenate([W, Wres[None]], axis=0)
    xw = _compute_xw(node_feats, w_all, bres, nrels, 2000)
    xw_flat = xw.reshape((nrels + 1) * n, d_out)
    res = xw_flat[nrels * n:]

    gidx = _compute_gidx(etype, src, n)

    sc = _make_sc_edge_kernel(n, d_out, e, 80)
    part = sc(gidx, dst, norm.reshape(e), xw_flat)

    return _combine_bn(part, res, bias, gamma, beta, n, d_out)


# speedup vs baseline: 12.6397x; 1.0770x over previous
import functools

import jax
import jax.numpy as jnp
from jax import lax
from jax.experimental import pallas as pl
from jax.experimental.pallas import tpu as pltpu
from jax.experimental.pallas import tpu_sc as plsc

_EPS = 1e-5
_NC = 2
_NS = 16
_LANES = 16


def _xw_body(nrels, x_ref, w_ref, bres_ref, out_ref):
    r = pl.program_id(0)
    t = jnp.dot(x_ref[...], w_ref[0], preferred_element_type=jnp.float32)
    t = jnp.where(r == nrels, jnp.maximum(t + bres_ref[...], 0.0), t)
    out_ref[0] = t


def _compute_xw(x, w_all, bres, nrels, block_n):
    n, d_in = x.shape
    rp1, _, d_out = w_all.shape
    grid = (rp1, n // block_n)
    return pl.pallas_call(
        functools.partial(_xw_body, nrels),
        grid=grid,
        in_specs=[
            pl.BlockSpec((block_n, d_in), lambda r, b: (b, 0)),
            pl.BlockSpec((1, d_in, d_out), lambda r, b: (r, 0, 0)),
            pl.BlockSpec((1, d_out), lambda r, b: (0, 0)),
        ],
        out_specs=pl.BlockSpec((1, block_n, d_out), lambda r, b: (r, b, 0)),
        out_shape=jax.ShapeDtypeStruct((rp1, n, d_out), jnp.float32),
    )(x, w_all, bres.reshape(1, d_out))


def _gidx_body(n, et_ref, src_ref, out_ref):
    out_ref[...] = et_ref[...] * n + src_ref[...]


def _compute_gidx(etype, src, n):
    e = etype.shape[0]
    cols = 128
    rows = e // cols
    out = pl.pallas_call(
        functools.partial(_gidx_body, n),
        out_shape=jax.ShapeDtypeStruct((rows, cols), jnp.int32),
    )(etype.reshape(rows, cols), src.reshape(rows, cols))
    return out.reshape(e)


def _make_sc_edge_kernel(n, d, e_pad, chunk):
    nw = _NC * _NS
    epw = e_pad // nw
    steps = epw // chunk
    assert steps % 2 == 0
    unit = 40
    units = n // unit
    rounds = (units + _NS - 1) // _NS

    def body(gidx_hbm, dst_hbm, norm_hbm, xw_hbm, out_hbm,
             iall_v, nall_v, rows0_v, rows1_v, dbuf0_v, dbuf1_v, zbuf_v,
             agg_sh, semg0, semg1, semd0, semd1, sems0, sems1):
        c = lax.axis_index("c")
        s = lax.axis_index("s")

        zv = jnp.zeros((_LANES,), jnp.float32)

        def zrow(i, _):
            for j in range(d // _LANES):
                zbuf_v[i, pl.ds(j * _LANES, _LANES)] = zv
            return 0

        lax.fori_loop(0, unit, zrow, 0)
        for k in range(rounds):
            u = s + k * _NS

            @pl.when(u < units)
            def _():
                pltpu.sync_copy(zbuf_v, agg_sh.at[pl.ds(u * unit, unit)])
        plsc.subcore_barrier()

        wid = s * _NC + c
        off0 = wid * epw
        pltpu.sync_copy(gidx_hbm.at[pl.ds(off0, epw)], iall_v)
        pltpu.sync_copy(norm_hbm.at[pl.ds(off0, epw)], nall_v)

        def start_gather(t, buf, sem):
            pltpu.async_copy(
                xw_hbm.at[iall_v.at[pl.ds(t * chunk, chunk)]], buf, sem)

        def wait_gather(buf, sem):
            pltpu.make_async_copy(xw_hbm.at[pl.ds(0, chunk)], buf, sem).wait()

        def start_dst(t, dbuf, sem):
            pltpu.async_copy(dst_hbm.at[pl.ds(off0 + t * chunk, chunk)],
                             dbuf, sem)

        def wait_dst(dbuf, sem):
            pltpu.make_async_copy(dst_hbm.at[pl.ds(0, chunk)], dbuf,
                                  sem).wait()

        def scale(t, buf):
            for g in range(chunk // _LANES):
                nv = nall_v[pl.ds(t * chunk + g * _LANES, _LANES)]
                for i in range(_LANES):
                    ee = g * _LANES + i
                    nb = nv[i]
                    for j in range(d // _LANES):
                        sl = pl.ds(j * _LANES, _LANES)
                        buf[ee, sl] = buf[ee, sl] * nb

        def start_scatter(buf, dbuf, sem):
            pltpu.async_copy(buf, agg_sh.at[dbuf], sem, add=True)

        def wait_scatter(buf, dbuf, sem):
            pltpu.make_async_copy(buf, agg_sh.at[dbuf], sem).wait()

        start_dst(0, dbuf0_v, semd0)
        start_dst(1, dbuf1_v, semd1)
        start_gather(0, rows0_v, semg0)
        start_gather(1, rows1_v, semg1)

        def step2(p, _):
            t0 = 2 * p
            wait_gather(rows0_v, semg0)
            scale(t0, rows0_v)
            wait_dst(dbuf0_v, semd0)
            start_scatter(rows0_v, dbuf0_v, sems0)

            wait_gather(rows1_v, semg1)
            scale(t0 + 1, rows1_v)
            wait_dst(dbuf1_v, semd1)
            start_scatter(rows1_v, dbuf1_v, sems1)

            wait_scatter(rows0_v, dbuf0_v, sems0)

            @pl.when(t0 + 2 < steps)
            def _():
                start_dst(t0 + 2, dbuf0_v, semd0)
                start_gather(t0 + 2, rows0_v, semg0)

            wait_scatter(rows1_v, dbuf1_v, sems1)

            @pl.when(t0 + 3 < steps)
            def _():
                start_dst(t0 + 3, dbuf1_v, semd1)
                start_gather(t0 + 3, rows1_v, semg1)

            return 0

        lax.fori_loop(0, steps // 2, step2, 0)
        plsc.subcore_barrier()

        for k in range(rounds):
            u = s + k * _NS

            @pl.when(u < units)
            def _():
                pltpu.sync_copy(agg_sh.at[pl.ds(u * unit, unit)], zbuf_v)
                pltpu.sync_copy(zbuf_v, out_hbm.at[pl.ds(c * n + u * unit,
                                                         unit)])

    mesh = plsc.VectorSubcoreMesh(core_axis_name="c", subcore_axis_name="s")
    return pl.kernel(
        body,
        out_type=jax.ShapeDtypeStruct((_NC * n, d), jnp.float32),
        mesh=mesh,
        scratch_types=[
            pltpu.VMEM((epw,), jnp.int32),
            pltpu.VMEM((epw,), jnp.float32),
            pltpu.VMEM((chunk, d), jnp.float32),
            pltpu.VMEM((chunk, d), jnp.float32),
            pltpu.VMEM((chunk,), jnp.int32),
            pltpu.VMEM((chunk,), jnp.int32),
            pltpu.VMEM((unit, d), jnp.float32),
            pltpu.VMEM_SHARED((n, d), jnp.float32),
            pltpu.SemaphoreType.DMA,
            pltpu.SemaphoreType.DMA,
            pltpu.SemaphoreType.DMA,
            pltpu.SemaphoreType.DMA,
            pltpu.SemaphoreType.DMA,
            pltpu.SemaphoreType.DMA,
        ],
    )


def _bn_body(n, eps, part_ref, res_ref, bias_ref, gamma_ref, beta_ref,
             out_ref):
    agg = part_ref[:n] + part_ref[n:]
    h = jnp.maximum(agg + bias_ref[...], 0.0)
    new = h + res_ref[...]
    mean = jnp.mean(new, axis=0, keepdims=True)
    var = jnp.mean((new - mean) * (new - mean), axis=0, keepdims=True)
    inv = lax.rsqrt(var + eps)
    out_ref[...] = (new - mean) * (inv * gamma_ref[...]) + beta_ref[...]


def _combine_bn(part, res, bias, gamma, beta, n, d):
    return pl.pallas_call(
        functools.partial(_bn_body, n, _EPS),
        out_shape=jax.ShapeDtypeStruct((n, d), jnp.float32),
    )(part, res, bias.reshape(1, d), gamma.reshape(1, d), beta.reshape(1, d))


def kernel(node_feats, edge_index, etype, norm, W, bias, Wres, bres, gamma,
           beta):
    n, d_in = node_feats.shape
    nrels, _, d_out = W.shape
    e = etype.shape[0]

    src = edge_index[0]
    dst = edge_index[1]

    w_all = jnp.concatenate([W, Wres[None]], axis=0)
    xw = _compute_xw(node_feats, w_all, bres, nrels, 2000)
    xw_flat = xw.reshape((nrels + 1) * n, d_out)
    res = xw_flat[nrels * n:]

    gidx = _compute_gidx(etype, src, n)

    chunk = 64
    nw = _NC * _NS
    quantum = nw * chunk * 2
    e_pad = ((e + quantum - 1) // quantum) * quantum
    pad = e_pad - e
    gidx_p = jnp.pad(gidx, (0, pad))
    dst_p = jnp.pad(dst, (0, pad))
    norm_p = jnp.pad(norm.reshape(e), (0, pad))

    sc = _make_sc_edge_kernel(n, d_out, e_pad, chunk)
    part = sc(gidx_p, dst_p, norm_p, xw_flat)

    return _combine_bn(part, res, bias, gamma, beta, n, d_out)
